# baseline (speedup 1.0000x reference)
import functools

import jax
import jax.numpy as jnp
from jax.experimental import pallas as pl
from jax.experimental.pallas import tpu as pltpu

_N = 65536
_B = 8
_T = 64
_K = 16
_R = 512
_C = 128
_CHUNK = 8192
_SEGROWS = 64
_SEG = _SEGROWS * _C
_NEG = float(jnp.finfo(jnp.float32).min)
_BIGI = 2147483647


def _fps_kernel(bid_ref, xyz_ref, sel_ref, valid_ref, counts_ref,
                mind2_ref):
    bid = bid_ref[...]

    cnts = []
    means = []
    for b in range(_B):
        mask_b = bid == b
        cnt_f = jnp.sum(mask_b.astype(jnp.float32))
        cnt_b = cnt_f.astype(jnp.int32)
        counts_ref[b] = cnt_b
        cnts.append(cnt_b)
        denom = jnp.maximum(cnt_f, 1.0)
        means.append([jnp.sum(jnp.where(mask_b, xyz_ref[c], 0.0)) / denom
                      for c in range(4)])
    active0 = tuple(cnts[b] > 0 for b in range(_B))
    offs = []
    o = jnp.int32(0)
    for b in range(_B):
        offs.append(o)
        o = o + cnts[b]

    mplanes = []
    for c in range(4):
        mp = jnp.zeros((_R, _C), jnp.float32)
        for b in range(_B):
            mp = jnp.where(bid == b, means[b][c], mp)
        mplanes.append(mp)
    d2m = ((xyz_ref[0] - mplanes[0]) ** 2
           + (xyz_ref[1] - mplanes[1]) ** 2
           + (xyz_ref[2] - mplanes[2]) ** 2
           + (xyz_ref[3] - mplanes[3]) ** 2)
    act_plane = jnp.zeros((_R, _C), jnp.float32)
    for b in range(_B):
        act_plane = jnp.where((bid == b) & active0[b], 1.0, act_plane)
    mind2_ref[...] = jnp.where(act_plane > 0, d2m, _NEG)

    lane = jax.lax.broadcasted_iota(jnp.int32, (1, _C), 1)
    gip = (jax.lax.broadcasted_iota(jnp.int32, (_R, _C), 0) * _C
           + jax.lax.broadcasted_iota(jnp.int32, (_R, _C), 1))
    rio = jax.lax.broadcasted_iota(jnp.int32, (_R, 1), 0)

    rmask_b, brows_b = [], []
    for b in range(_B):
        off, cnt = offs[b], cnts[b]
        end = off + cnt
        rmask_b.append((rio * _C >= off) & ((rio + 1) * _C <= end))
        rows = []
        for r in (off // _C, jnp.minimum(end // _C, _R - 1)):
            gir = r * _C + lane
            rows.append((r, (gir >= off) & (gir < end), gir))
        brows_b.append(rows)

    def _merge(bv, bi, cm, ci):
        bi = jnp.where(cm > bv, ci,
                       jnp.where(cm == bv, jnp.minimum(ci, bi), bi))
        return jnp.maximum(bv, cm), bi

    def body(t, active):
        sc = mind2_ref[...]
        rm = jnp.max(sc, axis=1, keepdims=True)
        ra = jnp.min(jnp.where(sc == rm, gip, _BIGI), axis=1,
                     keepdims=True)
        sels = []
        for b in range(_B):
            bv = jnp.max(jnp.where(rmask_b[b], rm, _NEG))
            bi = jnp.min(jnp.where(rmask_b[b] & (rm == bv), ra, _BIGI))
            for r, lm, gir in brows_b[b]:
                lo = mind2_ref[pl.ds(r, 1), :]
                mlo = jnp.where(lm, lo, _NEG)
                cm = jnp.max(mlo)
                ci = jnp.min(jnp.where(mlo == cm, gir, _BIGI))
                bv, bi = _merge(bv, bi, cm, ci)
            sels.append(bi)

        new_active = []
        cxs = []
        cxrow = [jnp.zeros((_R, 1), jnp.float32) for _ in range(4)]
        pvrow = jnp.zeros((_R, 1), jnp.float32)
        actrow = jnp.ones((_R, 1), jnp.float32)
        for b in range(_B):
            sel_b = sels[b]
            valid_b = active[b]
            a_b = valid_b & ((t + 1) < cnts[b])
            new_active.append(a_b)
            rs = jnp.minimum(sel_b, _N - 1)
            r = rs // _C
            col = rs % _C
            cx = []
            for c in range(4):
                rowv = xyz_ref[c, pl.ds(r, 1), :]
                cx.append(jnp.sum(jnp.where(lane == col, rowv, 0.0)))
            cxs.append(cx)
            for c in range(4):
                cxrow[c] = jnp.where(rmask_b[b], cx[c], cxrow[c])
            pvrow = jnp.where(rmask_b[b] & valid_b, 1.0, pvrow)
            actrow = jnp.where(rmask_b[b],
                               jnp.where(a_b, 1.0, 0.0), actrow)

        md = mind2_ref[...]
        d2n = ((xyz_ref[0] - cxrow[0]) ** 2 + (xyz_ref[1] - cxrow[1]) ** 2
               + (xyz_ref[2] - cxrow[2]) ** 2 + (xyz_ref[3] - cxrow[3]) ** 2)
        upd = jnp.where(t == 0, d2n, jnp.minimum(md, d2n))
        nm = jnp.where(pvrow > 0, upd, md)
        mind2_ref[...] = jnp.where(actrow > 0, nm, _NEG)

        for b in range(_B):
            sel_b = sels[b]
            valid_b = active[b]
            a_b = new_active[b]
            cx = cxs[b]
            for r, lm, gir in brows_b[b]:
                mrow = mind2_ref[pl.ds(r, 1), :]
                d2r = ((xyz_ref[0, pl.ds(r, 1), :] - cx[0]) ** 2
                       + (xyz_ref[1, pl.ds(r, 1), :] - cx[1]) ** 2
                       + (xyz_ref[2, pl.ds(r, 1), :] - cx[2]) ** 2
                       + (xyz_ref[3, pl.ds(r, 1), :] - cx[3]) ** 2)
                updr = jnp.where(t == 0, d2r, jnp.minimum(mrow, d2r))
                nmr = jnp.where(lm & valid_b, updr, mrow)
                nmr = jnp.where(lm & jnp.logical_not(a_b), _NEG, nmr)
                mind2_ref[pl.ds(r, 1), :] = nmr
            rs = jnp.minimum(sel_b, _N - 1)
            r = rs // _C
            col = rs % _C
            mrow = mind2_ref[pl.ds(r, 1), :]
            mind2_ref[pl.ds(r, 1), :] = jnp.where(
                (lane == col) & valid_b, _NEG, mrow)
            sel_ref[t, b] = jnp.where(valid_b, sel_b, jnp.int32(_N))
            valid_ref[t, b] = valid_b.astype(jnp.int32)
        return tuple(new_active)

    jax.lax.fori_loop(0, _T, body, active0, unroll=False)


def _knn_kernel(q_ref, xyz_ref, counts_ref, offs_ref, kidx_ref, kval_ref,
                topv_ref, topi_ref):
    b = pl.program_id(0)
    off = offs_ref[b]
    cnt = counts_ref[b]
    q = q_ref[0]
    qq = jnp.sum(q * q, axis=1, keepdims=True)
    topv_ref[...] = jnp.full((_T, _K), jnp.inf, jnp.float32)
    topi_ref[...] = jnp.zeros((_T, _K), jnp.int32)
    fc = off // _CHUNK
    nch = jnp.where(cnt > 0, (off + cnt - 1) // _CHUNK - fc + 1, 0)

    def body(i, _):
        start = pl.multiple_of((fc + i) * _CHUNK, _CHUNK)
        xs = xyz_ref[:, pl.ds(start, _CHUNK)]
        xx = jnp.sum(xs * xs, axis=0, keepdims=True)
        qx = jnp.dot(q, xs, preferred_element_type=jnp.float32)
        d2 = qq + xx - 2.0 * qx
        gidx = start + jax.lax.broadcasted_iota(jnp.int32, (_T, _CHUNK), 1)
        live = (gidx >= off) & (gidx < off + cnt)
        d2 = jnp.where(live, d2, jnp.inf)
        topv = topv_ref[...]
        topi = topi_ref[...]
        newv = []
        newi = []
        for _ in range(_K):
            m = jnp.minimum(jnp.min(d2, axis=1, keepdims=True),
                            jnp.min(topv, axis=1, keepdims=True))
            pick_d = jnp.min(jnp.where(d2 == m, gidx, _BIGI), axis=1,
                             keepdims=True)
            pick_t = jnp.min(jnp.where(topv == m, topi, _BIGI), axis=1,
                             keepdims=True)
            pick = jnp.minimum(pick_d, pick_t)
            d2 = jnp.where((d2 == m) & (gidx == pick), jnp.inf, d2)
            topv = jnp.where((topv == m) & (topi == pick), jnp.inf, topv)
            newv.append(m)
            newi.append(pick)
        topv_ref[...] = jnp.concatenate(newv, axis=1)
        topi_ref[...] = jnp.concatenate(newi, axis=1)
        return 0

    jax.lax.fori_loop(0, nch, body, 0, unroll=False)
    kidx_ref[0] = topi_ref[...]
    kval_ref[0] = topv_ref[...]


def _head_kernel(g_ref, kv_ref, qv_ref, w0_ref, b0_ref, w1_ref, b1_ref,
                 w2_ref, b2_ref, wn0_ref, bn0_ref, wn1_ref, bn1_ref,
                 out_ref):
    h = jnp.maximum(
        jnp.dot(g_ref[...], w0_ref[...], preferred_element_type=jnp.float32)
        + b0_ref[...], 0.0)
    h = jnp.maximum(
        jnp.dot(h, w1_ref[...], preferred_element_type=jnp.float32)
        + b1_ref[...], 0.0)
    pf = (jnp.dot(h, w2_ref[...], preferred_element_type=jnp.float32)
          + b2_ref[...])
    pf = jnp.where(kv_ref[...] > 0, pf, _NEG)
    pooled = pf[0:512, :]
    for j in range(1, _K):
        pooled = jnp.maximum(pooled, pf[512 * j:512 * (j + 1), :])
    a = jnp.maximum(
        jnp.dot(pooled, wn0_ref[...], preferred_element_type=jnp.float32)
        + bn0_ref[...], 0.0)
    a = (jnp.dot(a, wn1_ref[...], preferred_element_type=jnp.float32)
         + bn1_ref[...])
    out_ref[...] = jnp.where(qv_ref[...] > 0, a, 0.0)


@functools.partial(jax.jit, static_argnames=())
def kernel(coords, features, batch_ids, W0, b0, W1, b1, W2, b2,
           Wn0, bn0, Wn1, bn1):
    f32 = jnp.float32
    time_col = features[:, -1:]
    xyzt = jnp.concatenate([coords, time_col], axis=-1)

    xyz_planes = xyzt.T.reshape(4, _R, _C)
    sel_rows, valid_rows, counts = pl.pallas_call(
        _fps_kernel,
        in_specs=[pl.BlockSpec(memory_space=pltpu.VMEM),
                  pl.BlockSpec(memory_space=pltpu.VMEM)],
        out_specs=(pl.BlockSpec(memory_space=pltpu.SMEM),
                   pl.BlockSpec(memory_space=pltpu.SMEM),
                   pl.BlockSpec(memory_space=pltpu.SMEM)),
        out_shape=(jax.ShapeDtypeStruct((_T, _B), jnp.int32),
                   jax.ShapeDtypeStruct((_T, _B), jnp.int32),
                   jax.ShapeDtypeStruct((_B,), jnp.int32)),
        scratch_shapes=[pltpu.VMEM((_R, _C), f32)],
    )(batch_ids.reshape(_R, _C), xyz_planes)
    offsets = jnp.concatenate(
        [jnp.zeros((1,), counts.dtype), jnp.cumsum(counts[:-1])])

    sel = sel_rows.T
    valid = valid_rows.T.astype(bool)
    valid_flat = valid.reshape(-1)
    safe_flat = jnp.where(valid_flat, sel.reshape(-1), 0)
    q = xyzt[safe_flat]
    centroids = jnp.where(valid_flat[:, None], q, 0.0).reshape(_B, _T, 4)

    kidx, kval = pl.pallas_call(
        _knn_kernel,
        grid=(_B,),
        in_specs=[
            pl.BlockSpec((1, _T, 4), lambda b: (b, 0, 0)),
            pl.BlockSpec((4, _N), lambda b: (0, 0)),
            pl.BlockSpec(memory_space=pltpu.SMEM),
            pl.BlockSpec(memory_space=pltpu.SMEM),
        ],
        out_specs=(pl.BlockSpec((1, _T, _K), lambda b: (b, 0, 0)),
                   pl.BlockSpec((1, _T, _K), lambda b: (b, 0, 0))),
        out_shape=(jax.ShapeDtypeStruct((_B, _T, _K), jnp.int32),
                   jax.ShapeDtypeStruct((_B, _T, _K), f32)),
        scratch_shapes=[pltpu.VMEM((_T, _K), f32),
                        pltpu.VMEM((_T, _K), jnp.int32)],
    )(q.reshape(_B, _T, 4), xyzt.T.reshape(4, _N),
      counts.astype(jnp.int32), offsets.astype(jnp.int32))

    kidx2 = kidx.reshape(_B * _T, _K)
    kvalid = kval.reshape(_B * _T, _K) < jnp.inf
    gather_rows = jnp.minimum(kidx2.T.reshape(-1), _N - 1)
    G = features[gather_rows]
    kvm = kvalid.T.reshape(-1, 1).astype(f32)
    qvm = valid_flat[:, None].astype(f32)

    tokens512 = pl.pallas_call(
        _head_kernel,
        out_shape=jax.ShapeDtypeStruct((_B * _T, 256), f32),
    )(G, kvm, qvm, W0, b0.reshape(1, -1), W1, b1.reshape(1, -1),
      W2, b2.reshape(1, -1), Wn0, bn0.reshape(1, -1),
      Wn1, bn1.reshape(1, -1))

    tokens = tokens512.reshape(_B, _T, 256)
    return tokens, centroids, valid


# speedup vs baseline: 1.0418x; 1.0418x over previous
import functools

import jax
import jax.numpy as jnp
from jax.experimental import pallas as pl
from jax.experimental.pallas import tpu as pltpu

_N = 65536
_B = 8
_T = 64
_K = 16
_R = 512
_C = 128
_CHUNK = 4096
_NEG = float(jnp.finfo(jnp.float32).min)
_BIGI = 2147483647


def _fps_kernel(bid_ref, xyz_ref, sel_ref, valid_ref, counts_ref,
                mind2_ref):
    bid = bid_ref[...]

    cnts = []
    means = []
    for b in range(_B):
        mask_b = bid == b
        cnt_f = jnp.sum(mask_b.astype(jnp.float32))
        cnt_b = cnt_f.astype(jnp.int32)
        counts_ref[b] = cnt_b
        cnts.append(cnt_b)
        denom = jnp.maximum(cnt_f, 1.0)
        means.append([jnp.sum(jnp.where(mask_b, xyz_ref[c], 0.0)) / denom
                      for c in range(4)])
    active0 = tuple(cnts[b] > 0 for b in range(_B))
    offs = []
    o = jnp.int32(0)
    for b in range(_B):
        offs.append(o)
        o = o + cnts[b]

    mplanes = []
    for c in range(4):
        mp = jnp.zeros((_R, _C), jnp.float32)
        for b in range(_B):
            mp = jnp.where(bid == b, means[b][c], mp)
        mplanes.append(mp)
    d2m = ((xyz_ref[0] - mplanes[0]) ** 2
           + (xyz_ref[1] - mplanes[1]) ** 2
           + (xyz_ref[2] - mplanes[2]) ** 2
           + (xyz_ref[3] - mplanes[3]) ** 2)
    act_plane = jnp.zeros((_R, _C), jnp.float32)
    for b in range(_B):
        act_plane = jnp.where((bid == b) & active0[b], 1.0, act_plane)
    mind2_ref[...] = jnp.where(act_plane > 0, d2m, _NEG)

    lane = jax.lax.broadcasted_iota(jnp.int32, (1, _C), 1)
    gip = (jax.lax.broadcasted_iota(jnp.int32, (_R, _C), 0) * _C
           + jax.lax.broadcasted_iota(jnp.int32, (_R, _C), 1))
    rio = jax.lax.broadcasted_iota(jnp.int32, (_R, 1), 0)

    rmask_b, brows_b = [], []
    for b in range(_B):
        off, cnt = offs[b], cnts[b]
        end = off + cnt
        rmask_b.append((rio * _C >= off) & ((rio + 1) * _C <= end))
        rows = []
        for r in (off // _C, jnp.minimum(end // _C, _R - 1)):
            gir = r * _C + lane
            rows.append((r, (gir >= off) & (gir < end), gir))
        brows_b.append(rows)

    def _merge(bv, bi, cm, ci):
        bi = jnp.where(cm > bv, ci,
                       jnp.where(cm == bv, jnp.minimum(ci, bi), bi))
        return jnp.maximum(bv, cm), bi

    def body(t, active):
        sc = mind2_ref[...]
        rm = jnp.max(sc, axis=1, keepdims=True)
        ra = jnp.min(jnp.where(sc == rm, gip, _BIGI), axis=1,
                     keepdims=True)
        sels = []
        for b in range(_B):
            bv = jnp.max(jnp.where(rmask_b[b], rm, _NEG))
            bi = jnp.min(jnp.where(rmask_b[b] & (rm == bv), ra, _BIGI))
            for r, lm, gir in brows_b[b]:
                lo = mind2_ref[pl.ds(r, 1), :]
                mlo = jnp.where(lm, lo, _NEG)
                cm = jnp.max(mlo)
                ci = jnp.min(jnp.where(mlo == cm, gir, _BIGI))
                bv, bi = _merge(bv, bi, cm, ci)
            sels.append(bi)

        new_active = []
        cxs = []
        cxrow = [jnp.zeros((_R, 1), jnp.float32) for _ in range(4)]
        pvrow = jnp.zeros((_R, 1), jnp.float32)
        actrow = jnp.ones((_R, 1), jnp.float32)
        for b in range(_B):
            sel_b = sels[b]
            valid_b = active[b]
            a_b = valid_b & ((t + 1) < cnts[b])
            new_active.append(a_b)
            rs = jnp.minimum(sel_b, _N - 1)
            r = rs // _C
            col = rs % _C
            cx = []
            for c in range(4):
                rowv = xyz_ref[c, pl.ds(r, 1), :]
                cx.append(jnp.sum(jnp.where(lane == col, rowv, 0.0)))
            cxs.append(cx)
            for c in range(4):
                cxrow[c] = jnp.where(rmask_b[b], cx[c], cxrow[c])
            pvrow = jnp.where(rmask_b[b] & valid_b, 1.0, pvrow)
            actrow = jnp.where(rmask_b[b],
                               jnp.where(a_b, 1.0, 0.0), actrow)

        md = mind2_ref[...]
        d2n = ((xyz_ref[0] - cxrow[0]) ** 2 + (xyz_ref[1] - cxrow[1]) ** 2
               + (xyz_ref[2] - cxrow[2]) ** 2 + (xyz_ref[3] - cxrow[3]) ** 2)
        upd = jnp.where(t == 0, d2n, jnp.minimum(md, d2n))
        nm = jnp.where(pvrow > 0, upd, md)
        mind2_ref[...] = jnp.where(actrow > 0, nm, _NEG)

        for b in range(_B):
            sel_b = sels[b]
            valid_b = active[b]
            a_b = new_active[b]
            cx = cxs[b]
            for r, lm, gir in brows_b[b]:
                mrow = mind2_ref[pl.ds(r, 1), :]
                d2r = ((xyz_ref[0, pl.ds(r, 1), :] - cx[0]) ** 2
                       + (xyz_ref[1, pl.ds(r, 1), :] - cx[1]) ** 2
                       + (xyz_ref[2, pl.ds(r, 1), :] - cx[2]) ** 2
                       + (xyz_ref[3, pl.ds(r, 1), :] - cx[3]) ** 2)
                updr = jnp.where(t == 0, d2r, jnp.minimum(mrow, d2r))
                nmr = jnp.where(lm & valid_b, updr, mrow)
                nmr = jnp.where(lm & jnp.logical_not(a_b), _NEG, nmr)
                mind2_ref[pl.ds(r, 1), :] = nmr
            rs = jnp.minimum(sel_b, _N - 1)
            r = rs // _C
            col = rs % _C
            mrow = mind2_ref[pl.ds(r, 1), :]
            mind2_ref[pl.ds(r, 1), :] = jnp.where(
                (lane == col) & valid_b, _NEG, mrow)
            sel_ref[t, b] = jnp.where(valid_b, sel_b, jnp.int32(_N))
            valid_ref[t, b] = valid_b.astype(jnp.int32)
        return tuple(new_active)

    jax.lax.fori_loop(0, _T, body, active0, unroll=False)


def _knn_kernel(q_ref, xyz_ref, counts_ref, offs_ref, kidx_ref, kval_ref,
                topv_ref, topi_ref):
    b = pl.program_id(0)
    off = offs_ref[b]
    cnt = counts_ref[b]
    q = q_ref[0]
    qq = jnp.sum(q * q, axis=1, keepdims=True)
    topv_ref[...] = jnp.full((_T, _K), jnp.inf, jnp.float32)
    topi_ref[...] = jnp.zeros((_T, _K), jnp.int32)
    fc = off // _CHUNK
    nch = jnp.where(cnt > 0, (off + cnt - 1) // _CHUNK - fc + 1, 0)

    def body(i, _):
        start = pl.multiple_of((fc + i) * _CHUNK, _CHUNK)
        xs = xyz_ref[:, pl.ds(start, _CHUNK)]
        xx = jnp.sum(xs * xs, axis=0, keepdims=True)
        qx = jnp.dot(q, xs, preferred_element_type=jnp.float32)
        d2 = qq + xx - 2.0 * qx
        gidx = start + jax.lax.broadcasted_iota(jnp.int32, (_T, _CHUNK), 1)
        live = (gidx >= off) & (gidx < off + cnt)
        d2 = jnp.where(live, d2, jnp.inf)
        topv = topv_ref[...]
        topi = topi_ref[...]
        newv = []
        newi = []
        for _ in range(_K):
            m = jnp.minimum(jnp.min(d2, axis=1, keepdims=True),
                            jnp.min(topv, axis=1, keepdims=True))
            pick_d = jnp.min(jnp.where(d2 == m, gidx, _BIGI), axis=1,
                             keepdims=True)
            pick_t = jnp.min(jnp.where(topv == m, topi, _BIGI), axis=1,
                             keepdims=True)
            pick = jnp.minimum(pick_d, pick_t)
            d2 = jnp.where((d2 == m) & (gidx == pick), jnp.inf, d2)
            topv = jnp.where((topv == m) & (topi == pick), jnp.inf, topv)
            newv.append(m)
            newi.append(pick)
        topv_ref[...] = jnp.concatenate(newv, axis=1)
        topi_ref[...] = jnp.concatenate(newi, axis=1)
        return 0

    jax.lax.fori_loop(0, nch, body, 0, unroll=False)
    kidx_ref[0] = topi_ref[...]
    kval_ref[0] = topv_ref[...]


def _head_kernel(g_ref, kv_ref, qv_ref, w0_ref, b0_ref, w1_ref, b1_ref,
                 w2_ref, b2_ref, wn0_ref, bn0_ref, wn1_ref, bn1_ref,
                 out_ref):
    h = jnp.maximum(
        jnp.dot(g_ref[...], w0_ref[...], preferred_element_type=jnp.float32)
        + b0_ref[...], 0.0)
    h = jnp.maximum(
        jnp.dot(h, w1_ref[...], preferred_element_type=jnp.float32)
        + b1_ref[...], 0.0)
    pf = (jnp.dot(h, w2_ref[...], preferred_element_type=jnp.float32)
          + b2_ref[...])
    pf = jnp.where(kv_ref[...] > 0, pf, _NEG)
    pooled = pf[0:512, :]
    for j in range(1, _K):
        pooled = jnp.maximum(pooled, pf[512 * j:512 * (j + 1), :])
    a = jnp.maximum(
        jnp.dot(pooled, wn0_ref[...], preferred_element_type=jnp.float32)
        + bn0_ref[...], 0.0)
    a = (jnp.dot(a, wn1_ref[...], preferred_element_type=jnp.float32)
         + bn1_ref[...])
    out_ref[...] = jnp.where(qv_ref[...] > 0, a, 0.0)


@functools.partial(jax.jit, static_argnames=())
def kernel(coords, features, batch_ids, W0, b0, W1, b1, W2, b2,
           Wn0, bn0, Wn1, bn1):
    f32 = jnp.float32
    time_col = features[:, -1:]
    xyzt = jnp.concatenate([coords, time_col], axis=-1)

    xyz_planes = xyzt.T.reshape(4, _R, _C)
    sel_rows, valid_rows, counts = pl.pallas_call(
        _fps_kernel,
        in_specs=[pl.BlockSpec(memory_space=pltpu.VMEM),
                  pl.BlockSpec(memory_space=pltpu.VMEM)],
        out_specs=(pl.BlockSpec(memory_space=pltpu.SMEM),
                   pl.BlockSpec(memory_space=pltpu.SMEM),
                   pl.BlockSpec(memory_space=pltpu.SMEM)),
        out_shape=(jax.ShapeDtypeStruct((_T, _B), jnp.int32),
                   jax.ShapeDtypeStruct((_T, _B), jnp.int32),
                   jax.ShapeDtypeStruct((_B,), jnp.int32)),
        scratch_shapes=[pltpu.VMEM((_R, _C), f32)],
    )(batch_ids.reshape(_R, _C), xyz_planes)
    offsets = jnp.concatenate(
        [jnp.zeros((1,), counts.dtype), jnp.cumsum(counts[:-1])])

    sel = sel_rows.T
    valid = valid_rows.T.astype(bool)
    valid_flat = valid.reshape(-1)
    safe_flat = jnp.where(valid_flat, sel.reshape(-1), 0)
    q = xyzt[safe_flat]
    centroids = jnp.where(valid_flat[:, None], q, 0.0).reshape(_B, _T, 4)

    kidx, kval = pl.pallas_call(
        _knn_kernel,
        grid=(_B,),
        in_specs=[
            pl.BlockSpec((1, _T, 4), lambda b: (b, 0, 0)),
            pl.BlockSpec((4, _N), lambda b: (0, 0)),
            pl.BlockSpec(memory_space=pltpu.SMEM),
            pl.BlockSpec(memory_space=pltpu.SMEM),
        ],
        out_specs=(pl.BlockSpec((1, _T, _K), lambda b: (b, 0, 0)),
                   pl.BlockSpec((1, _T, _K), lambda b: (b, 0, 0))),
        out_shape=(jax.ShapeDtypeStruct((_B, _T, _K), jnp.int32),
                   jax.ShapeDtypeStruct((_B, _T, _K), f32)),
        scratch_shapes=[pltpu.VMEM((_T, _K), f32),
                        pltpu.VMEM((_T, _K), jnp.int32)],
    )(q.reshape(_B, _T, 4), xyzt.T.reshape(4, _N),
      counts.astype(jnp.int32), offsets.astype(jnp.int32))

    kidx2 = kidx.reshape(_B * _T, _K)
    kvalid = kval.reshape(_B * _T, _K) < jnp.inf
    gather_rows = jnp.minimum(kidx2.T.reshape(-1), _N - 1)
    G = features[gather_rows]
    kvm = kvalid.T.reshape(-1, 1).astype(f32)
    qvm = valid_flat[:, None].astype(f32)

    tokens512 = pl.pallas_call(
        _head_kernel,
        out_shape=jax.ShapeDtypeStruct((_B * _T, 256), f32),
    )(G, kvm, qvm, W0, b0.reshape(1, -1), W1, b1.reshape(1, -1),
      W2, b2.reshape(1, -1), Wn0, bn0.reshape(1, -1),
      Wn1, bn1.reshape(1, -1))

    tokens = tokens512.reshape(_B, _T, 256)
    return tokens, centroids, valid
